# Initial kernel scaffold; baseline (speedup 1.0000x reference)
import jax
import jax.numpy as jnp
from jax.experimental import pallas as pl


def kernel(x, edge_index, W, b, u):
    raise NotImplementedError("write your pallas kernel here")


# speedup vs baseline: 5.7077x; 5.7077x over previous
import functools

import jax
import jax.numpy as jnp
from jax import lax
from jax.experimental import pallas as pl
from jax.experimental.pallas import tpu as pltpu
from jax.experimental.pallas import tpu_sc as plsc

N = 10000
E = 160000
D = 256
DH = 128
NS = 16
NC = 2
EPT = E // NS
K = 125
NCH = EPT // K
RPT = 640
N_PAD = 10240
ZR = 128

_sc_mesh = plsc.VectorSubcoreMesh(core_axis_name="c", subcore_axis_name="s")


def _sc_body(x0, x1, row3, col3, zrows, zdeg, ones_h,
             out_agg, out_deg,
             rowl, coll, rowsv, onesv, agg_s, deg_s):
    cid = lax.axis_index("c")
    sid = lax.axis_index("s")

    pltpu.sync_copy(row3.at[sid], rowl)
    pltpu.sync_copy(col3.at[sid], coll)
    pltpu.sync_copy(ones_h, onesv)

    @pl.loop(0, RPT // ZR)
    def _(j):
        pltpu.sync_copy(zrows, agg_s.at[pl.ds(sid * RPT + j * ZR, ZR)])

    pltpu.sync_copy(zdeg, deg_s.at[pl.ds(sid * RPT, RPT)])
    plsc.subcore_barrier()

    @pl.loop(0, NCH)
    def _(ch):
        idxc = coll.at[ch]
        idxr = rowl.at[ch]

        @pl.when(cid == 0)
        def _():
            pltpu.sync_copy(x0.at[idxc], rowsv)

        @pl.when(cid == 1)
        def _():
            pltpu.sync_copy(x1.at[idxc], rowsv)

        pltpu.sync_copy(rowsv, agg_s.at[idxr], add=True)

        @pl.when(cid == 0)
        def _():
            pltpu.sync_copy(onesv, deg_s.at[idxr], add=True)

    plsc.subcore_barrier()

    @pl.when(sid < NS - 1)
    def _():
        pltpu.sync_copy(agg_s.at[pl.ds(sid * RPT, RPT)],
                        out_agg.at[cid, pl.ds(sid * RPT, RPT)])

    @pl.when(sid == NS - 1)
    def _():
        pltpu.sync_copy(agg_s.at[pl.ds((NS - 1) * RPT, N - (NS - 1) * RPT)],
                        out_agg.at[cid, pl.ds((NS - 1) * RPT, N - (NS - 1) * RPT)])

    @pl.when(jnp.logical_and(cid == 0, sid == 0))
    def _():
        pltpu.sync_copy(deg_s, out_deg)


_sc_fn = functools.partial(
    pl.kernel,
    out_type=[jax.ShapeDtypeStruct((NC, N, DH), jnp.float32),
              jax.ShapeDtypeStruct((N_PAD,), jnp.float32)],
    mesh=_sc_mesh,
    scratch_types=[
        pltpu.VMEM((NCH, K), jnp.int32),
        pltpu.VMEM((NCH, K), jnp.int32),
        pltpu.VMEM((K, DH), jnp.float32),
        pltpu.VMEM((K,), jnp.float32),
        pltpu.VMEM_SHARED((N_PAD, DH), jnp.float32),
        pltpu.VMEM_SHARED((N_PAD,), jnp.float32),
    ],
)(_sc_body)


BM = 512


def _tc_body(agg_ref, deg_ref, w_ref, wt_ref, u_ref, b_ref, out_ref):
    hi = lax.Precision.HIGHEST
    wt = wt_ref[...]
    u = u_ref[...]
    v = jnp.dot(wt, u, preferred_element_type=jnp.float32, precision=hi)
    v = v / (jnp.sqrt(jnp.sum(v * v)) + 1e-12)
    wv = jnp.dot(w_ref[...], v, preferred_element_type=jnp.float32, precision=hi)
    n1 = jnp.sqrt(jnp.sum(wv * wv)) + 1e-12
    sigma = jnp.sum((wv / n1) * wv)

    a = jnp.concatenate([agg_ref[0], agg_ref[1]], axis=1)
    dg = jnp.maximum(deg_ref[...], 1.0)
    a = a / dg
    mm = jnp.dot(a, wt, preferred_element_type=jnp.float32, precision=hi)
    out_ref[...] = mm / sigma + b_ref[...]


def _tc_fn(agg, deg, W, WT, u, b):
    grid = (pl.cdiv(N, BM),)
    return pl.pallas_call(
        _tc_body,
        grid=grid,
        in_specs=[
            pl.BlockSpec((NC, BM, DH), lambda i: (0, i, 0)),
            pl.BlockSpec((BM, 1), lambda i: (i, 0)),
            pl.BlockSpec((D, D), lambda i: (0, 0)),
            pl.BlockSpec((D, D), lambda i: (0, 0)),
            pl.BlockSpec((D, 1), lambda i: (0, 0)),
            pl.BlockSpec((1, D), lambda i: (0, 0)),
        ],
        out_specs=pl.BlockSpec((BM, D), lambda i: (i, 0)),
        out_shape=jax.ShapeDtypeStruct((N, D), jnp.float32),
    )(agg, deg, W, WT, u, b)


def kernel(x, edge_index, W, b, u):
    x0 = x[:, :DH]
    x1 = x[:, DH:]
    row3 = edge_index[0].reshape(NS, NCH, K)
    col3 = edge_index[1].reshape(NS, NCH, K)
    zrows = jnp.zeros((ZR, DH), jnp.float32)
    zdeg = jnp.zeros((RPT,), jnp.float32)
    ones_h = jnp.ones((K,), jnp.float32)
    agg, deg = _sc_fn(x0, x1, row3, col3, zrows, zdeg, ones_h)
    out = _tc_fn(agg, deg[:N].reshape(N, 1), W, W.T, u.reshape(D, 1),
                 b.reshape(1, D))
    return out


# speedup vs baseline: 6.5859x; 1.1539x over previous
import dataclasses
import functools

import jax
import jax.numpy as jnp
from jax import lax
from jax.experimental import pallas as pl
from jax.experimental.pallas import tpu as pltpu
from jax.experimental.pallas import tpu_sc as plsc

N = 10000
E = 160000
D = 256
DH = 128
NS = 16
NC = 2
EPT = E // NS
K = 125
NCH = EPT // K
RPT = 640
N_PAD = 10240
ZR = 128
RING = 16

_sc_mesh = plsc.VectorSubcoreMesh(core_axis_name="c", subcore_axis_name="s")

_sc_params = pltpu.CompilerParams()
if "needs_layout_passes" in pltpu.CompilerParams.__dataclass_fields__:
    _sc_params = dataclasses.replace(_sc_params, needs_layout_passes=False)


def _sc_body(xh, row3, col3a, col3b, zrows, zdeg, ones_h,
             out_agg, out_deg0, out_deg1,
             rowl, coll, rows2, onesv, agg_s, deg_s, sem2):
    cid = lax.axis_index("c")
    sid = lax.axis_index("s")

    def gstart(ch, par):
        pltpu.async_copy(xh.at[coll.at[ch]], rows2.at[par], sem2.at[par])

    def gwait(par):
        pltpu.make_async_copy(xh.at[coll.at[0]], rows2.at[par],
                              sem2.at[par]).wait()

    def refill(base, half):
        pltpu.sync_copy(row3.at[sid, pl.ds(base, RING // 2)],
                        rowl.at[pl.ds(half, RING // 2)])

        @pl.when(cid == 0)
        def _():
            pltpu.sync_copy(col3a.at[sid, pl.ds(base, RING // 2)],
                            coll.at[pl.ds(half, RING // 2)])

        @pl.when(cid == 1)
        def _():
            pltpu.sync_copy(col3b.at[sid, pl.ds(base, RING // 2)],
                            coll.at[pl.ds(half, RING // 2)])

    refill(0, 0)
    refill(RING // 2, RING // 2)
    pltpu.sync_copy(ones_h, onesv)

    @pl.loop(0, RPT // ZR)
    def _(j):
        pltpu.sync_copy(zrows, agg_s.at[pl.ds(sid * RPT + j * ZR, ZR)])

    pltpu.sync_copy(zdeg, deg_s.at[pl.ds(sid * RPT, RPT)])
    plsc.subcore_barrier()

    @pl.loop(0, NCH)
    def _(ch):
        par = lax.rem(ch, 2)
        slot = lax.rem(ch, RING)
        deg_mine = jnp.where(cid == 0, ch < NCH // 2, ch >= NCH // 2)

        @pl.when(ch == 0)
        def _():
            gstart(0, 0)

        gwait(par)

        @pl.when(ch < NCH - 1)
        def _():
            gstart(lax.rem(ch + 1, RING), 1 - par)

        @pl.when(jnp.logical_and(lax.rem(ch, RING // 2) == 0,
                                 jnp.logical_and(ch >= RING // 2,
                                                 ch <= NCH - RING)))
        def _():
            half = pl.multiple_of(lax.rem(ch + RING // 2, RING), RING // 2)
            refill(pl.multiple_of(ch + RING // 2, RING // 2), half)

        pltpu.sync_copy(rows2.at[par], agg_s.at[rowl.at[slot]], add=True)

        @pl.when(deg_mine)
        def _():
            pltpu.sync_copy(onesv, deg_s.at[rowl.at[slot]], add=True)

    plsc.subcore_barrier()

    @pl.when(sid < NS - 1)
    def _():
        pltpu.sync_copy(agg_s.at[pl.ds(sid * RPT, RPT)],
                        out_agg.at[cid, pl.ds(sid * RPT, RPT)])

    @pl.when(sid == NS - 1)
    def _():
        pltpu.sync_copy(agg_s.at[pl.ds((NS - 1) * RPT, N - (NS - 1) * RPT)],
                        out_agg.at[cid, pl.ds((NS - 1) * RPT, N - (NS - 1) * RPT)])

    @pl.when(jnp.logical_and(cid == 0, sid == 0))
    def _():
        pltpu.sync_copy(deg_s, out_deg0)

    @pl.when(jnp.logical_and(cid == 1, sid == 0))
    def _():
        pltpu.sync_copy(deg_s, out_deg1)


_sc_fn = functools.partial(
    pl.kernel,
    out_type=[jax.ShapeDtypeStruct((NC, N, DH), jnp.float32),
              jax.ShapeDtypeStruct((N_PAD,), jnp.float32),
              jax.ShapeDtypeStruct((N_PAD,), jnp.float32)],
    mesh=_sc_mesh,
    compiler_params=_sc_params,
    scratch_types=[
        pltpu.VMEM((RING, K), jnp.int32),
        pltpu.VMEM((RING, K), jnp.int32),
        pltpu.VMEM((2, K, DH), jnp.float32),
        pltpu.VMEM((K,), jnp.float32),
        pltpu.VMEM_SHARED((N_PAD, DH), jnp.float32),
        pltpu.VMEM_SHARED((N_PAD,), jnp.float32),
        pltpu.SemaphoreType.DMA((2,)),
    ],
)(_sc_body)


BM = 512


def _tc_body(agg_ref, d0_ref, d1_ref, w_ref, wt_ref, u_ref, b_ref, out_ref):
    hi = lax.Precision.HIGHEST
    wt = wt_ref[...]
    u = u_ref[...]
    v = jnp.dot(wt, u, preferred_element_type=jnp.float32, precision=hi)
    v = v / (jnp.sqrt(jnp.sum(v * v)) + 1e-12)
    wv = jnp.dot(w_ref[...], v, preferred_element_type=jnp.float32, precision=hi)
    n1 = jnp.sqrt(jnp.sum(wv * wv)) + 1e-12
    sigma = jnp.sum((wv / n1) * wv)

    a = jnp.concatenate([agg_ref[0], agg_ref[1]], axis=1)
    dg = jnp.maximum(d0_ref[...] + d1_ref[...], 1.0)
    a = a / dg
    mm = jnp.dot(a, wt, preferred_element_type=jnp.float32, precision=hi)
    out_ref[...] = mm / sigma + b_ref[...]


def _tc_fn(agg, d0, d1, W, WT, u, b):
    grid = (pl.cdiv(N, BM),)
    return pl.pallas_call(
        _tc_body,
        grid=grid,
        in_specs=[
            pl.BlockSpec((NC, BM, DH), lambda i: (0, i, 0)),
            pl.BlockSpec((BM, 1), lambda i: (i, 0)),
            pl.BlockSpec((BM, 1), lambda i: (i, 0)),
            pl.BlockSpec((D, D), lambda i: (0, 0)),
            pl.BlockSpec((D, D), lambda i: (0, 0)),
            pl.BlockSpec((D, 1), lambda i: (0, 0)),
            pl.BlockSpec((1, D), lambda i: (0, 0)),
        ],
        out_specs=pl.BlockSpec((BM, D), lambda i: (i, 0)),
        out_shape=jax.ShapeDtypeStruct((N, D), jnp.float32),
    )(agg, d0, d1, W, WT, u, b)


def kernel(x, edge_index, W, b, u):
    xh = jnp.concatenate([x[:, :DH], x[:, DH:]], axis=0)
    row3 = edge_index[0].reshape(NS, NCH, K)
    col3a = edge_index[1].reshape(NS, NCH, K)
    col3b = col3a + N
    zrows = jnp.zeros((ZR, DH), jnp.float32)
    zdeg = jnp.zeros((RPT,), jnp.float32)
    ones_h = jnp.ones((K,), jnp.float32)
    agg, deg0, deg1 = _sc_fn(xh, row3, col3a, col3b, zrows, zdeg, ones_h)
    out = _tc_fn(agg, deg0[:N].reshape(N, 1), deg1[:N].reshape(N, 1),
                 W, W.T, u.reshape(D, 1), b.reshape(1, D))
    return out


# speedup vs baseline: 6.7225x; 1.0208x over previous
import dataclasses
import functools

import jax
import jax.numpy as jnp
from jax import lax
from jax.experimental import pallas as pl
from jax.experimental.pallas import tpu as pltpu
from jax.experimental.pallas import tpu_sc as plsc

N = 10000
E = 160000
D = 256
DH = 128
NS = 16
NC = 2
EPT = E // NS
K = 125
NCH = EPT // K
RPT = 640
N_PAD = 10240
ZR = 128
RING = 16

_sc_mesh = plsc.VectorSubcoreMesh(core_axis_name="c", subcore_axis_name="s")

_sc_params = pltpu.CompilerParams()
if "needs_layout_passes" in pltpu.CompilerParams.__dataclass_fields__:
    _sc_params = dataclasses.replace(_sc_params, needs_layout_passes=False)


def _sc_body(xh, row3, col3a, col3b, zrows, zdeg, ones_h,
             out_agg, out_deg0, out_deg1,
             rowl, coll, rows2, onesv, agg_s, deg_s, sem2, ssem2):
    cid = lax.axis_index("c")
    sid = lax.axis_index("s")

    def gstart(ch, par):
        pltpu.async_copy(xh.at[coll.at[ch]], rows2.at[par], sem2.at[par])

    def gwait(par):
        pltpu.make_async_copy(xh.at[coll.at[0]], rows2.at[par],
                              sem2.at[par]).wait()

    def sstart(slot, par):
        pltpu.async_copy(rows2.at[par], agg_s.at[rowl.at[slot]],
                         ssem2.at[par], add=True)

    def swait(par):
        pltpu.make_async_copy(rows2.at[par], agg_s.at[rowl.at[0]],
                              ssem2.at[par]).wait()

    def refill(base, half):
        pltpu.sync_copy(row3.at[sid, pl.ds(base, RING // 2)],
                        rowl.at[pl.ds(half, RING // 2)])

        @pl.when(cid == 0)
        def _():
            pltpu.sync_copy(col3a.at[sid, pl.ds(base, RING // 2)],
                            coll.at[pl.ds(half, RING // 2)])

        @pl.when(cid == 1)
        def _():
            pltpu.sync_copy(col3b.at[sid, pl.ds(base, RING // 2)],
                            coll.at[pl.ds(half, RING // 2)])

    refill(0, 0)
    refill(RING // 2, RING // 2)
    pltpu.sync_copy(ones_h, onesv)

    @pl.loop(0, RPT // ZR)
    def _(j):
        pltpu.sync_copy(zrows, agg_s.at[pl.ds(sid * RPT + j * ZR, ZR)])

    pltpu.sync_copy(zdeg, deg_s.at[pl.ds(sid * RPT, RPT)])
    plsc.subcore_barrier()

    @pl.loop(0, NCH)
    def _(ch):
        par = lax.rem(ch, 2)
        slot = lax.rem(ch, RING)
        deg_mine = jnp.where(cid == 0, ch < NCH // 2, ch >= NCH // 2)

        @pl.when(ch == 0)
        def _():
            gstart(0, 0)

        gwait(par)
        sstart(slot, par)

        @pl.when(ch >= 1)
        def _():
            swait(1 - par)

        @pl.when(ch < NCH - 1)
        def _():
            gstart(lax.rem(ch + 1, RING), 1 - par)

        @pl.when(jnp.logical_and(lax.rem(ch, RING // 2) == 0,
                                 jnp.logical_and(ch >= RING // 2,
                                                 ch <= NCH - RING)))
        def _():
            half = pl.multiple_of(lax.rem(ch + RING // 2, RING), RING // 2)
            refill(pl.multiple_of(ch + RING // 2, RING // 2), half)

        @pl.when(deg_mine)
        def _():
            pltpu.sync_copy(onesv, deg_s.at[rowl.at[slot]], add=True)

    swait(lax.rem(NCH - 1, 2))
    plsc.subcore_barrier()

    @pl.when(sid < NS - 1)
    def _():
        pltpu.sync_copy(agg_s.at[pl.ds(sid * RPT, RPT)],
                        out_agg.at[cid, pl.ds(sid * RPT, RPT)])

    @pl.when(sid == NS - 1)
    def _():
        pltpu.sync_copy(agg_s.at[pl.ds((NS - 1) * RPT, N - (NS - 1) * RPT)],
                        out_agg.at[cid, pl.ds((NS - 1) * RPT, N - (NS - 1) * RPT)])

    @pl.when(jnp.logical_and(cid == 0, sid == 0))
    def _():
        pltpu.sync_copy(deg_s, out_deg0)

    @pl.when(jnp.logical_and(cid == 1, sid == 0))
    def _():
        pltpu.sync_copy(deg_s, out_deg1)


_sc_fn = functools.partial(
    pl.kernel,
    out_type=[jax.ShapeDtypeStruct((NC, N, DH), jnp.float32),
              jax.ShapeDtypeStruct((N_PAD,), jnp.float32),
              jax.ShapeDtypeStruct((N_PAD,), jnp.float32)],
    mesh=_sc_mesh,
    compiler_params=_sc_params,
    scratch_types=[
        pltpu.VMEM((RING, K), jnp.int32),
        pltpu.VMEM((RING, K), jnp.int32),
        pltpu.VMEM((2, K, DH), jnp.float32),
        pltpu.VMEM((K,), jnp.float32),
        pltpu.VMEM_SHARED((N_PAD, DH), jnp.float32),
        pltpu.VMEM_SHARED((N_PAD,), jnp.float32),
        pltpu.SemaphoreType.DMA((2,)),
        pltpu.SemaphoreType.DMA((2,)),
    ],
)(_sc_body)


BM = 512


def _tc_body(agg_ref, d0_ref, d1_ref, w_ref, wt_ref, u_ref, b_ref, out_ref):
    hi = lax.Precision.HIGHEST
    wt = wt_ref[...]
    u = u_ref[...]
    v = jnp.dot(wt, u, preferred_element_type=jnp.float32, precision=hi)
    v = v / (jnp.sqrt(jnp.sum(v * v)) + 1e-12)
    wv = jnp.dot(w_ref[...], v, preferred_element_type=jnp.float32, precision=hi)
    n1 = jnp.sqrt(jnp.sum(wv * wv)) + 1e-12
    sigma = jnp.sum((wv / n1) * wv)

    a = jnp.concatenate([agg_ref[0], agg_ref[1]], axis=1)
    dg = jnp.maximum(d0_ref[...] + d1_ref[...], 1.0)
    a = a / dg
    mm = jnp.dot(a, wt, preferred_element_type=jnp.float32, precision=hi)
    out_ref[...] = mm / sigma + b_ref[...]


def _tc_fn(agg, d0, d1, W, WT, u, b):
    grid = (pl.cdiv(N, BM),)
    return pl.pallas_call(
        _tc_body,
        grid=grid,
        in_specs=[
            pl.BlockSpec((NC, BM, DH), lambda i: (0, i, 0)),
            pl.BlockSpec((BM, 1), lambda i: (i, 0)),
            pl.BlockSpec((BM, 1), lambda i: (i, 0)),
            pl.BlockSpec((D, D), lambda i: (0, 0)),
            pl.BlockSpec((D, D), lambda i: (0, 0)),
            pl.BlockSpec((D, 1), lambda i: (0, 0)),
            pl.BlockSpec((1, D), lambda i: (0, 0)),
        ],
        out_specs=pl.BlockSpec((BM, D), lambda i: (i, 0)),
        out_shape=jax.ShapeDtypeStruct((N, D), jnp.float32),
    )(agg, d0, d1, W, WT, u, b)


def kernel(x, edge_index, W, b, u):
    xh = jnp.concatenate([x[:, :DH], x[:, DH:]], axis=0)
    row3 = edge_index[0].reshape(NS, NCH, K)
    col3a = edge_index[1].reshape(NS, NCH, K)
    col3b = col3a + N
    zrows = jnp.zeros((ZR, DH), jnp.float32)
    zdeg = jnp.zeros((RPT,), jnp.float32)
    ones_h = jnp.ones((K,), jnp.float32)
    agg, deg0, deg1 = _sc_fn(xh, row3, col3a, col3b, zrows, zdeg, ones_h)
    out = _tc_fn(agg, deg0[:N].reshape(N, 1), deg1[:N].reshape(N, 1),
                 W, W.T, u.reshape(D, 1), b.reshape(1, D))
    return out


# speedup vs baseline: 6.7592x; 1.0055x over previous
import dataclasses
import functools

import jax
import jax.numpy as jnp
from jax import lax
from jax.experimental import pallas as pl
from jax.experimental.pallas import tpu as pltpu
from jax.experimental.pallas import tpu_sc as plsc

N = 10000
E = 160000
D = 256
DH = 128
NS = 16
NC = 2
EPT = E // NS
K = 125
NCH = EPT // K
RPT = 640
N_PAD = 10240
ZR = 128
RING = 16

_sc_mesh = plsc.VectorSubcoreMesh(core_axis_name="c", subcore_axis_name="s")

_sc_params = pltpu.CompilerParams()
if "needs_layout_passes" in pltpu.CompilerParams.__dataclass_fields__:
    _sc_params = dataclasses.replace(_sc_params, needs_layout_passes=False)


def _sc_body(xh, row3, col3a, col3b, zrows, zdeg, ones_h,
             out_agg, out_deg0, out_deg1,
             rowl, coll, rows2, onesv, agg_s, deg_s, sem2, ssem2):
    cid = lax.axis_index("c")
    sid = lax.axis_index("s")

    def gstart(ch, par):
        pltpu.async_copy(xh.at[coll.at[ch]], rows2.at[par], sem2.at[par])

    def gwait(par):
        pltpu.make_async_copy(xh.at[coll.at[0]], rows2.at[par],
                              sem2.at[par]).wait()

    def sstart(slot, par):
        pltpu.async_copy(rows2.at[par], agg_s.at[rowl.at[slot]],
                         ssem2.at[par], add=True)

    def swait(par):
        pltpu.make_async_copy(rows2.at[par], agg_s.at[rowl.at[0]],
                              ssem2.at[par]).wait()

    def refill(base, half):
        pltpu.sync_copy(row3.at[sid, pl.ds(base, RING // 2)],
                        rowl.at[pl.ds(half, RING // 2)])

        @pl.when(cid == 0)
        def _():
            pltpu.sync_copy(col3a.at[sid, pl.ds(base, RING // 2)],
                            coll.at[pl.ds(half, RING // 2)])

        @pl.when(cid == 1)
        def _():
            pltpu.sync_copy(col3b.at[sid, pl.ds(base, RING // 2)],
                            coll.at[pl.ds(half, RING // 2)])

    refill(0, 0)
    refill(RING // 2, RING // 2)
    pltpu.sync_copy(ones_h, onesv)

    @pl.loop(0, RPT // ZR)
    def _(j):
        pltpu.sync_copy(zrows, agg_s.at[pl.ds(sid * RPT + j * ZR, ZR)])

    pltpu.sync_copy(zdeg, deg_s.at[pl.ds(sid * RPT, RPT)])
    plsc.subcore_barrier()

    @pl.loop(0, NCH)
    def _(ch):
        par = lax.rem(ch, 2)
        slot = lax.rem(ch, RING)
        deg_mine = jnp.where(cid == 0, ch < NCH // 2, ch >= NCH // 2)

        @pl.when(ch == 0)
        def _():
            gstart(0, 0)

        gwait(par)
        sstart(slot, par)

        @pl.when(ch >= 1)
        def _():
            swait(1 - par)

        @pl.when(ch < NCH - 1)
        def _():
            gstart(lax.rem(ch + 1, RING), 1 - par)

        @pl.when(jnp.logical_and(lax.rem(ch, RING // 2) == 0,
                                 jnp.logical_and(ch >= RING // 2,
                                                 ch <= NCH - RING)))
        def _():
            half = pl.multiple_of(lax.rem(ch + RING // 2, RING), RING // 2)
            refill(pl.multiple_of(ch + RING // 2, RING // 2), half)

        @pl.when(deg_mine)
        def _():
            pltpu.sync_copy(onesv, deg_s.at[rowl.at[slot]], add=True)

    swait(lax.rem(NCH - 1, 2))
    plsc.subcore_barrier()

    @pl.when(sid < NS - 1)
    def _():
        pltpu.sync_copy(agg_s.at[pl.ds(sid * RPT, RPT)],
                        out_agg.at[cid, pl.ds(sid * RPT, RPT)])

    @pl.when(sid == NS - 1)
    def _():
        pltpu.sync_copy(agg_s.at[pl.ds((NS - 1) * RPT, N - (NS - 1) * RPT)],
                        out_agg.at[cid, pl.ds((NS - 1) * RPT, N - (NS - 1) * RPT)])

    @pl.when(jnp.logical_and(cid == 0, sid == 0))
    def _():
        pltpu.sync_copy(deg_s, out_deg0)

    @pl.when(jnp.logical_and(cid == 1, sid == 0))
    def _():
        pltpu.sync_copy(deg_s, out_deg1)


_sc_fn = functools.partial(
    pl.kernel,
    out_type=[jax.ShapeDtypeStruct((NC, N, DH), jnp.float32),
              jax.ShapeDtypeStruct((N_PAD,), jnp.float32),
              jax.ShapeDtypeStruct((N_PAD,), jnp.float32)],
    mesh=_sc_mesh,
    compiler_params=_sc_params,
    scratch_types=[
        pltpu.VMEM((RING, K), jnp.int32),
        pltpu.VMEM((RING, K), jnp.int32),
        pltpu.VMEM((2, K, DH), jnp.float32),
        pltpu.VMEM((K,), jnp.float32),
        pltpu.VMEM_SHARED((N_PAD, DH), jnp.float32),
        pltpu.VMEM_SHARED((N_PAD,), jnp.float32),
        pltpu.SemaphoreType.DMA((2,)),
        pltpu.SemaphoreType.DMA((2,)),
    ],
)(_sc_body)


BM = 512


def _tc_body(agg_ref, d0_ref, d1_ref, w_ref, wt_ref, u_ref, b_ref, out_ref):
    hi = lax.Precision.HIGHEST
    wt = wt_ref[...]
    u = u_ref[...]
    v = jnp.dot(wt, u, preferred_element_type=jnp.float32, precision=hi)
    v = v / (jnp.sqrt(jnp.sum(v * v)) + 1e-12)
    wv = jnp.dot(w_ref[...], v, preferred_element_type=jnp.float32, precision=hi)
    n1 = jnp.sqrt(jnp.sum(wv * wv)) + 1e-12
    sigma = jnp.sum((wv / n1) * wv)

    a = jnp.concatenate([agg_ref[0], agg_ref[1]], axis=1)
    dg = jnp.maximum(d0_ref[...] + d1_ref[...], 1.0)
    a = a / dg
    mm = jnp.dot(a, wt, preferred_element_type=jnp.float32, precision=hi)
    out_ref[...] = mm / sigma + b_ref[...]


def _tc_fn(agg, d0, d1, W, WT, u, b):
    grid = (pl.cdiv(N, BM),)
    return pl.pallas_call(
        _tc_body,
        grid=grid,
        in_specs=[
            pl.BlockSpec((NC, BM, DH), lambda i: (0, i, 0)),
            pl.BlockSpec((BM, 1), lambda i: (i, 0)),
            pl.BlockSpec((BM, 1), lambda i: (i, 0)),
            pl.BlockSpec((D, D), lambda i: (0, 0)),
            pl.BlockSpec((D, D), lambda i: (0, 0)),
            pl.BlockSpec((D, 1), lambda i: (0, 0)),
            pl.BlockSpec((1, D), lambda i: (0, 0)),
        ],
        out_specs=pl.BlockSpec((BM, D), lambda i: (i, 0)),
        out_shape=jax.ShapeDtypeStruct((N, D), jnp.float32),
    )(agg, d0, d1, W, WT, u, b)


def kernel(x, edge_index, W, b, u):
    xh = x.reshape(NC * N, DH)
    row3 = edge_index[0].reshape(NS, NCH, K)
    col3a = (edge_index[1] * 2).reshape(NS, NCH, K)
    col3b = col3a + 1
    zrows = jnp.zeros((ZR, DH), jnp.float32)
    zdeg = jnp.zeros((RPT,), jnp.float32)
    ones_h = jnp.ones((K,), jnp.float32)
    agg, deg0, deg1 = _sc_fn(xh, row3, col3a, col3b, zrows, zdeg, ones_h)
    out = _tc_fn(agg, deg0[:N].reshape(N, 1), deg1[:N].reshape(N, 1),
                 W, W.T, u.reshape(D, 1), b.reshape(1, D))
    return out


# speedup vs baseline: 7.0369x; 1.0411x over previous
import dataclasses
import functools

import jax
import jax.numpy as jnp
from jax import lax
from jax.experimental import pallas as pl
from jax.experimental.pallas import tpu as pltpu
from jax.experimental.pallas import tpu_sc as plsc

N = 10000
E = 160000
D = 256
DH = 128
NS = 16
NC = 2
EPT = E // NS
K = 125
NCH = EPT // K
RPT = 640
N_PAD = 10240
ZR = 128
RING = 16

_sc_mesh = plsc.VectorSubcoreMesh(core_axis_name="c", subcore_axis_name="s")

_sc_params = pltpu.CompilerParams()
if "needs_layout_passes" in pltpu.CompilerParams.__dataclass_fields__:
    _sc_params = dataclasses.replace(_sc_params, needs_layout_passes=False)


def _sc_body(xh, row3, col3a, col3b, zrows, zdeg, ones_h,
             out_agg, out_deg0, out_deg1,
             rowl, coll, rows2, onesv, agg_s, deg_s, sem2, ssem2):
    cid = lax.axis_index("c")
    sid = lax.axis_index("s")

    def gstart(ch, par):
        pltpu.async_copy(xh.at[coll.at[ch]], rows2.at[par], sem2.at[par])

    def gwait(par):
        pltpu.make_async_copy(xh.at[coll.at[0]], rows2.at[par],
                              sem2.at[par]).wait()

    def sstart(slot, par):
        pltpu.async_copy(rows2.at[par], agg_s.at[rowl.at[slot]],
                         ssem2.at[par], add=True)

    def swait(par):
        pltpu.make_async_copy(rows2.at[par], agg_s.at[rowl.at[0]],
                              ssem2.at[par]).wait()

    def refill(base, half):
        pltpu.sync_copy(row3.at[sid, pl.ds(base, RING // 2)],
                        rowl.at[pl.ds(half, RING // 2)])

        @pl.when(cid == 0)
        def _():
            pltpu.sync_copy(col3a.at[sid, pl.ds(base, RING // 2)],
                            coll.at[pl.ds(half, RING // 2)])

        @pl.when(cid == 1)
        def _():
            pltpu.sync_copy(col3b.at[sid, pl.ds(base, RING // 2)],
                            coll.at[pl.ds(half, RING // 2)])

    refill(0, 0)
    refill(RING // 2, RING // 2)
    pltpu.sync_copy(ones_h, onesv)

    @pl.loop(0, RPT // ZR)
    def _(j):
        pltpu.sync_copy(zrows, agg_s.at[pl.ds(sid * RPT + j * ZR, ZR)])

    pltpu.sync_copy(zdeg, deg_s.at[pl.ds(sid * RPT, RPT)])
    plsc.subcore_barrier()

    @pl.loop(0, NCH)
    def _(ch):
        par = lax.rem(ch, 2)
        slot = lax.rem(ch, RING)
        deg_mine = jnp.where(cid == 0, ch < NCH // 2, ch >= NCH // 2)

        @pl.when(ch == 0)
        def _():
            gstart(0, 0)

        gwait(par)
        sstart(slot, par)

        @pl.when(ch >= 1)
        def _():
            swait(1 - par)

        @pl.when(ch < NCH - 1)
        def _():
            gstart(lax.rem(ch + 1, RING), 1 - par)

        @pl.when(jnp.logical_and(lax.rem(ch, RING // 2) == 0,
                                 jnp.logical_and(ch >= RING // 2,
                                                 ch <= NCH - RING)))
        def _():
            half = pl.multiple_of(lax.rem(ch + RING // 2, RING), RING // 2)
            refill(pl.multiple_of(ch + RING // 2, RING // 2), half)

        @pl.when(deg_mine)
        def _():
            pltpu.sync_copy(onesv, deg_s.at[rowl.at[slot]], add=True)

    swait(lax.rem(NCH - 1, 2))
    plsc.subcore_barrier()

    @pl.when(sid < NS - 1)
    def _():
        pltpu.sync_copy(agg_s.at[pl.ds(sid * RPT, RPT)],
                        out_agg.at[cid, pl.ds(sid * RPT, RPT)])

    @pl.when(sid == NS - 1)
    def _():
        pltpu.sync_copy(agg_s.at[pl.ds((NS - 1) * RPT, N - (NS - 1) * RPT)],
                        out_agg.at[cid, pl.ds((NS - 1) * RPT, N - (NS - 1) * RPT)])

    @pl.when(jnp.logical_and(cid == 0, sid == 0))
    def _():
        pltpu.sync_copy(deg_s, out_deg0)

    @pl.when(jnp.logical_and(cid == 1, sid == 0))
    def _():
        pltpu.sync_copy(deg_s, out_deg1)


_sc_fn = functools.partial(
    pl.kernel,
    out_type=[jax.ShapeDtypeStruct((NC, N, DH), jnp.float32),
              jax.ShapeDtypeStruct((N_PAD,), jnp.float32),
              jax.ShapeDtypeStruct((N_PAD,), jnp.float32)],
    mesh=_sc_mesh,
    compiler_params=_sc_params,
    scratch_types=[
        pltpu.VMEM((RING, K), jnp.int32),
        pltpu.VMEM((RING, K), jnp.int32),
        pltpu.VMEM((2, K, DH), jnp.float32),
        pltpu.VMEM((K,), jnp.float32),
        pltpu.VMEM_SHARED((N_PAD, DH), jnp.float32),
        pltpu.VMEM_SHARED((N_PAD,), jnp.float32),
        pltpu.SemaphoreType.DMA((2,)),
        pltpu.SemaphoreType.DMA((2,)),
    ],
)(_sc_body)


BM = 512


def _tc_body(agg_ref, dinv_ref, w_ref, wt_ref, u_ref, b_ref, out_ref):
    hi = lax.Precision.HIGHEST
    wt = wt_ref[...]
    u = u_ref[...]
    v = jnp.dot(wt, u, preferred_element_type=jnp.float32, precision=hi)
    v = v / (jnp.sqrt(jnp.sum(v * v)) + 1e-12)
    wv = jnp.dot(w_ref[...], v, preferred_element_type=jnp.float32, precision=hi)
    n1 = jnp.sqrt(jnp.sum(wv * wv)) + 1e-12
    sigma = jnp.sum((wv / n1) * wv)

    a = jnp.concatenate([agg_ref[0], agg_ref[1]], axis=1)
    a = a * dinv_ref[...]
    mm = jnp.dot(a, wt, preferred_element_type=jnp.float32)
    out_ref[...] = mm / sigma + b_ref[...]


def _tc_fn(agg, dinv, W, WT, u, b):
    grid = (pl.cdiv(N, BM),)
    return pl.pallas_call(
        _tc_body,
        grid=grid,
        in_specs=[
            pl.BlockSpec((NC, BM, DH), lambda i: (0, i, 0)),
            pl.BlockSpec((BM, 1), lambda i: (i, 0)),
            pl.BlockSpec((D, D), lambda i: (0, 0)),
            pl.BlockSpec((D, D), lambda i: (0, 0)),
            pl.BlockSpec((D, 1), lambda i: (0, 0)),
            pl.BlockSpec((1, D), lambda i: (0, 0)),
        ],
        out_specs=pl.BlockSpec((BM, D), lambda i: (i, 0)),
        out_shape=jax.ShapeDtypeStruct((N, D), jnp.float32),
    )(agg, dinv, W, WT, u, b)


def kernel(x, edge_index, W, b, u):
    xh = x.reshape(NC * N, DH)
    row3 = edge_index[0].reshape(NS, NCH, K)
    col3a = (edge_index[1] * 2).reshape(NS, NCH, K)
    col3b = col3a + 1
    zrows = jnp.zeros((ZR, DH), jnp.float32)
    zdeg = jnp.zeros((RPT,), jnp.float32)
    ones_h = jnp.ones((K,), jnp.float32)
    agg, deg0, deg1 = _sc_fn(xh, row3, col3a, col3b, zrows, zdeg, ones_h)
    dinv = (1.0 / jnp.maximum(deg0[:N] + deg1[:N], 1.0)).reshape(N, 1)
    out = _tc_fn(agg, dinv,
                 W, W.T, u.reshape(D, 1), b.reshape(1, D))
    return out


# speedup vs baseline: 7.0468x; 1.0014x over previous
import dataclasses
import functools

import jax
import jax.numpy as jnp
from jax import lax
from jax.experimental import pallas as pl
from jax.experimental.pallas import tpu as pltpu
from jax.experimental.pallas import tpu_sc as plsc

N = 10000
E = 160000
D = 256
DH = 128
NS = 16
NC = 2
K = 128
NCHT = E // K
CPT = 80
CPT_LAST = NCHT - 15 * CPT
NCH_PAD = 1256
RPT = 640
N_PAD = 10240
ZR = 128
RING = 16

_sc_mesh = plsc.VectorSubcoreMesh(core_axis_name="c", subcore_axis_name="s")

_sc_params = pltpu.CompilerParams()
if "needs_layout_passes" in pltpu.CompilerParams.__dataclass_fields__:
    _sc_params = dataclasses.replace(_sc_params, needs_layout_passes=False)


def _sc_body(xh, row4, col4a, col4b, zrows, zdeg, ones_h,
             out_agg, out_deg0, out_deg1,
             rowl, coll, rows2, onesv, agg_s, deg_s, sem2, ssem2):
    cid = lax.axis_index("c")
    sid = lax.axis_index("s")
    nch = jnp.where(sid < NS - 1, CPT, CPT_LAST)
    b0 = sid * CPT

    def gstart(ch, par):
        pltpu.async_copy(xh.at[coll.at[ch]], rows2.at[par], sem2.at[par])

    def gwait(par):
        pltpu.make_async_copy(xh.at[coll.at[0]], rows2.at[par],
                              sem2.at[par]).wait()

    def sstart(slot, par):
        pltpu.async_copy(rows2.at[par], agg_s.at[rowl.at[slot]],
                         ssem2.at[par], add=True)

    def swait(par):
        pltpu.make_async_copy(rows2.at[par], agg_s.at[rowl.at[0]],
                              ssem2.at[par]).wait()

    def refill(base, half):
        pltpu.sync_copy(row4.at[pl.ds(b0 + base, RING // 2)],
                        rowl.at[pl.ds(half, RING // 2)])

        @pl.when(cid == 0)
        def _():
            pltpu.sync_copy(col4a.at[pl.ds(b0 + base, RING // 2)],
                            coll.at[pl.ds(half, RING // 2)])

        @pl.when(cid == 1)
        def _():
            pltpu.sync_copy(col4b.at[pl.ds(b0 + base, RING // 2)],
                            coll.at[pl.ds(half, RING // 2)])

    refill(0, 0)
    refill(RING // 2, RING // 2)
    pltpu.sync_copy(ones_h, onesv)

    @pl.loop(0, RPT // ZR)
    def _(j):
        pltpu.sync_copy(zrows, agg_s.at[pl.ds(sid * RPT + j * ZR, ZR)])

    pltpu.sync_copy(zdeg, deg_s.at[pl.ds(sid * RPT, RPT)])
    plsc.subcore_barrier()

    @pl.loop(0, nch)
    def _(ch):
        par = lax.rem(ch, 2)
        slot = lax.rem(ch, RING)
        deg_mine = jnp.where(cid == 0, ch < nch // 2, ch >= nch // 2)

        @pl.when(ch == 0)
        def _():
            gstart(0, 0)

        gwait(par)
        sstart(slot, par)

        @pl.when(ch >= 1)
        def _():
            swait(1 - par)

        @pl.when(ch < nch - 1)
        def _():
            gstart(lax.rem(ch + 1, RING), 1 - par)

        @pl.when(jnp.logical_and(lax.rem(ch, RING // 2) == 0,
                                 jnp.logical_and(ch >= RING // 2,
                                                 ch + RING // 2 < nch)))
        def _():
            half = pl.multiple_of(lax.rem(ch + RING // 2, RING), RING // 2)
            refill(pl.multiple_of(ch + RING // 2, RING // 2), half)

        @pl.when(deg_mine)
        def _():
            pltpu.sync_copy(onesv, deg_s.at[rowl.at[slot]], add=True)

    swait(lax.rem(nch - 1, 2))
    plsc.subcore_barrier()

    @pl.when(sid < NS - 1)
    def _():
        pltpu.sync_copy(agg_s.at[pl.ds(sid * RPT, RPT)],
                        out_agg.at[cid, pl.ds(sid * RPT, RPT)])

    @pl.when(sid == NS - 1)
    def _():
        pltpu.sync_copy(agg_s.at[pl.ds((NS - 1) * RPT, N - (NS - 1) * RPT)],
                        out_agg.at[cid, pl.ds((NS - 1) * RPT, N - (NS - 1) * RPT)])

    @pl.when(jnp.logical_and(cid == 0, sid == 0))
    def _():
        pltpu.sync_copy(deg_s, out_deg0)

    @pl.when(jnp.logical_and(cid == 1, sid == 0))
    def _():
        pltpu.sync_copy(deg_s, out_deg1)


_sc_fn = functools.partial(
    pl.kernel,
    out_type=[jax.ShapeDtypeStruct((NC, N, DH), jnp.float32),
              jax.ShapeDtypeStruct((N_PAD,), jnp.float32),
              jax.ShapeDtypeStruct((N_PAD,), jnp.float32)],
    mesh=_sc_mesh,
    compiler_params=_sc_params,
    scratch_types=[
        pltpu.VMEM((RING, K), jnp.int32),
        pltpu.VMEM((RING, K), jnp.int32),
        pltpu.VMEM((2, K, DH), jnp.float32),
        pltpu.VMEM((K,), jnp.float32),
        pltpu.VMEM_SHARED((N_PAD, DH), jnp.float32),
        pltpu.VMEM_SHARED((N_PAD,), jnp.float32),
        pltpu.SemaphoreType.DMA((2,)),
        pltpu.SemaphoreType.DMA((2,)),
    ],
)(_sc_body)


BM = 512


def _tc_body(agg_ref, dinv_ref, w_ref, wt_ref, u_ref, b_ref, out_ref):
    hi = lax.Precision.HIGHEST
    wt = wt_ref[...]
    u = u_ref[...]
    v = jnp.dot(wt, u, preferred_element_type=jnp.float32, precision=hi)
    v = v / (jnp.sqrt(jnp.sum(v * v)) + 1e-12)
    wv = jnp.dot(w_ref[...], v, preferred_element_type=jnp.float32, precision=hi)
    n1 = jnp.sqrt(jnp.sum(wv * wv)) + 1e-12
    sigma = jnp.sum((wv / n1) * wv)

    a = jnp.concatenate([agg_ref[0], agg_ref[1]], axis=1)
    a = a * dinv_ref[...]
    mm = jnp.dot(a, wt, preferred_element_type=jnp.float32)
    out_ref[...] = mm / sigma + b_ref[...]


def _tc_fn(agg, dinv, W, WT, u, b):
    grid = (pl.cdiv(N, BM),)
    return pl.pallas_call(
        _tc_body,
        grid=grid,
        in_specs=[
            pl.BlockSpec((NC, BM, DH), lambda i: (0, i, 0)),
            pl.BlockSpec((BM, 1), lambda i: (i, 0)),
            pl.BlockSpec((D, D), lambda i: (0, 0)),
            pl.BlockSpec((D, D), lambda i: (0, 0)),
            pl.BlockSpec((D, 1), lambda i: (0, 0)),
            pl.BlockSpec((1, D), lambda i: (0, 0)),
        ],
        out_specs=pl.BlockSpec((BM, D), lambda i: (i, 0)),
        out_shape=jax.ShapeDtypeStruct((N, D), jnp.float32),
    )(agg, dinv, W, WT, u, b)


def kernel(x, edge_index, W, b, u):
    xh = x.reshape(NC * N, DH)
    pad = (0, NCH_PAD * K - E)
    row4 = jnp.pad(edge_index[0], pad).reshape(NCH_PAD, K)
    col4a = jnp.pad(edge_index[1] * 2, pad).reshape(NCH_PAD, K)
    col4b = col4a + 1
    zrows = jnp.zeros((ZR, DH), jnp.float32)
    zdeg = jnp.zeros((RPT,), jnp.float32)
    ones_h = jnp.ones((K,), jnp.float32)
    agg, deg0, deg1 = _sc_fn(xh, row4, col4a, col4b, zrows, zdeg, ones_h)
    dinv = (1.0 / jnp.maximum(deg0[:N] + deg1[:N], 1.0)).reshape(N, 1)
    out = _tc_fn(agg, dinv,
                 W, W.T, u.reshape(D, 1), b.reshape(1, D))
    return out


# speedup vs baseline: 7.8887x; 1.1195x over previous
import dataclasses
import functools

import jax
import jax.numpy as jnp
from jax import lax
from jax.experimental import pallas as pl
from jax.experimental.pallas import tpu as pltpu
from jax.experimental.pallas import tpu_sc as plsc

N = 10000
E = 160000
D = 256
DH = 128
NS = 16
NC = 2
K = 128
NCHT = E // K
CPT = 80
CPT_LAST = NCHT - 15 * CPT
NCH_PAD = 1256
RPT = 640
N_PAD = 10240
ZR = 128
RING = 16

_sc_mesh = plsc.VectorSubcoreMesh(core_axis_name="c", subcore_axis_name="s")

_sc_params = pltpu.CompilerParams()
if "needs_layout_passes" in pltpu.CompilerParams.__dataclass_fields__:
    _sc_params = dataclasses.replace(_sc_params, needs_layout_passes=False)


def _sc_body(xh, row4, col4, zrows, zdeg, ones_h,
             out_agg, out_deg0, out_deg1,
             rowl, coll, rows2, onesv, agg_s, deg_s, sem2, ssem2):
    cid = lax.axis_index("c")
    sid = lax.axis_index("s")
    nch = jnp.where(sid < NS - 1, CPT, CPT_LAST)
    b0 = sid * CPT

    coff = pl.multiple_of(cid * DH, DH)

    def gstart(ch, par):
        pltpu.async_copy(xh.at[coll.at[ch], pl.ds(coff, DH)], rows2.at[par],
                         sem2.at[par])

    def gwait(par):
        pltpu.make_async_copy(xh.at[coll.at[0], pl.ds(coff, DH)],
                              rows2.at[par], sem2.at[par]).wait()

    def sstart(slot, par):
        pltpu.async_copy(rows2.at[par], agg_s.at[rowl.at[slot]],
                         ssem2.at[par], add=True)

    def swait(par):
        pltpu.make_async_copy(rows2.at[par], agg_s.at[rowl.at[0]],
                              ssem2.at[par]).wait()

    def refill(base, half):
        pltpu.sync_copy(row4.at[pl.ds(b0 + base, RING // 2)],
                        rowl.at[pl.ds(half, RING // 2)])

        pltpu.sync_copy(col4.at[pl.ds(b0 + base, RING // 2)],
                        coll.at[pl.ds(half, RING // 2)])

    refill(0, 0)
    refill(RING // 2, RING // 2)
    pltpu.sync_copy(ones_h, onesv)

    @pl.loop(0, RPT // ZR)
    def _(j):
        pltpu.sync_copy(zrows, agg_s.at[pl.ds(sid * RPT + j * ZR, ZR)])

    pltpu.sync_copy(zdeg, deg_s.at[pl.ds(sid * RPT, RPT)])
    plsc.subcore_barrier()

    @pl.loop(0, nch)
    def _(ch):
        par = lax.rem(ch, 2)
        slot = lax.rem(ch, RING)
        deg_mine = jnp.where(cid == 0, ch < nch // 2, ch >= nch // 2)

        @pl.when(ch == 0)
        def _():
            gstart(0, 0)

        gwait(par)
        sstart(slot, par)

        @pl.when(ch >= 1)
        def _():
            swait(1 - par)

        @pl.when(ch < nch - 1)
        def _():
            gstart(lax.rem(ch + 1, RING), 1 - par)

        @pl.when(jnp.logical_and(lax.rem(ch, RING // 2) == 0,
                                 jnp.logical_and(ch >= RING // 2,
                                                 ch + RING // 2 < nch)))
        def _():
            half = pl.multiple_of(lax.rem(ch + RING // 2, RING), RING // 2)
            refill(pl.multiple_of(ch + RING // 2, RING // 2), half)

        @pl.when(deg_mine)
        def _():
            pltpu.sync_copy(onesv, deg_s.at[rowl.at[slot]], add=True)

    swait(lax.rem(nch - 1, 2))
    plsc.subcore_barrier()

    @pl.when(sid < NS - 1)
    def _():
        pltpu.sync_copy(agg_s.at[pl.ds(sid * RPT, RPT)],
                        out_agg.at[cid, pl.ds(sid * RPT, RPT)])

    @pl.when(sid == NS - 1)
    def _():
        pltpu.sync_copy(agg_s.at[pl.ds((NS - 1) * RPT, N - (NS - 1) * RPT)],
                        out_agg.at[cid, pl.ds((NS - 1) * RPT, N - (NS - 1) * RPT)])

    @pl.when(jnp.logical_and(cid == 0, sid == 0))
    def _():
        pltpu.sync_copy(deg_s, out_deg0)

    @pl.when(jnp.logical_and(cid == 1, sid == 0))
    def _():
        pltpu.sync_copy(deg_s, out_deg1)


_sc_fn = functools.partial(
    pl.kernel,
    out_type=[jax.ShapeDtypeStruct((NC, N, DH), jnp.float32),
              jax.ShapeDtypeStruct((N_PAD,), jnp.float32),
              jax.ShapeDtypeStruct((N_PAD,), jnp.float32)],
    mesh=_sc_mesh,
    compiler_params=_sc_params,
    scratch_types=[
        pltpu.VMEM((RING, K), jnp.int32),
        pltpu.VMEM((RING, K), jnp.int32),
        pltpu.VMEM((2, K, DH), jnp.float32),
        pltpu.VMEM((K,), jnp.float32),
        pltpu.VMEM_SHARED((N_PAD, DH), jnp.float32),
        pltpu.VMEM_SHARED((N_PAD,), jnp.float32),
        pltpu.SemaphoreType.DMA((2,)),
        pltpu.SemaphoreType.DMA((2,)),
    ],
)(_sc_body)


BM = 512


def _tc_body(agg_ref, dinv_ref, w_ref, wt_ref, u_ref, b_ref, out_ref,
             sinv_ref):
    @pl.when(pl.program_id(0) == 0)
    def _():
        hi = lax.Precision.HIGHEST
        wt = wt_ref[...]
        u = u_ref[...]
        v = jnp.dot(wt, u, preferred_element_type=jnp.float32, precision=hi)
        v = v / (jnp.sqrt(jnp.sum(v * v)) + 1e-12)
        wv = jnp.dot(w_ref[...], v, preferred_element_type=jnp.float32,
                     precision=hi)
        n1 = jnp.sqrt(jnp.sum(wv * wv)) + 1e-12
        sinv_ref[0] = 1.0 / jnp.sum((wv / n1) * wv)

    a0 = agg_ref[0] * dinv_ref[...]
    a1 = agg_ref[1] * dinv_ref[...]
    mm = (jnp.dot(a0, wt_ref[:DH], preferred_element_type=jnp.float32)
          + jnp.dot(a1, wt_ref[DH:], preferred_element_type=jnp.float32))
    out_ref[...] = mm * sinv_ref[0] + b_ref[...]


def _tc_fn(agg, dinv, W, WT, u, b):
    grid = (pl.cdiv(N, BM),)
    return pl.pallas_call(
        _tc_body,
        grid=grid,
        in_specs=[
            pl.BlockSpec((NC, BM, DH), lambda i: (0, i, 0)),
            pl.BlockSpec((BM, 1), lambda i: (i, 0)),
            pl.BlockSpec((D, D), lambda i: (0, 0)),
            pl.BlockSpec((D, D), lambda i: (0, 0)),
            pl.BlockSpec((D, 1), lambda i: (0, 0)),
            pl.BlockSpec((1, D), lambda i: (0, 0)),
        ],
        out_specs=pl.BlockSpec((BM, D), lambda i: (i, 0)),
        out_shape=jax.ShapeDtypeStruct((N, D), jnp.float32),
        scratch_shapes=[pltpu.SMEM((1,), jnp.float32)],
    )(agg, dinv, W, WT, u, b)


def kernel(x, edge_index, W, b, u):
    pad = (0, NCH_PAD * K - E)
    row4 = jnp.pad(edge_index[0], pad).reshape(NCH_PAD, K)
    col4 = jnp.pad(edge_index[1], pad).reshape(NCH_PAD, K)
    zrows = jnp.zeros((ZR, DH), jnp.float32)
    zdeg = jnp.zeros((RPT,), jnp.float32)
    ones_h = jnp.ones((K,), jnp.float32)
    agg, deg0, deg1 = _sc_fn(x, row4, col4, zrows, zdeg, ones_h)
    dinv = (1.0 / jnp.maximum(deg0[:N] + deg1[:N], 1.0)).reshape(N, 1)
    out = _tc_fn(agg, dinv,
                 W, W.T, u.reshape(D, 1), b.reshape(1, D))
    return out
